# baseline (speedup 1.0000x reference)
import jax
import jax.numpy as jnp
from jax import lax
from jax.experimental import pallas as pl
from jax.experimental.pallas import tpu as pltpu
from jax.experimental.pallas import tpu_sc as plsc

N = 10000
NP = 10240
D = 128
E = 320000
EP = 327680
NTILES = 32
EPT = 10240
NCHUNK = 4
CH = 2560
SUBSL = NP // 16


def _mesh():
    return plsc.VectorSubcoreMesh(
        core_axis_name="c", subcore_axis_name="s", num_cores=2, num_subcores=16
    )


def _iota16():
    return lax.broadcasted_iota(jnp.int32, (16,), 0)


def _leaky_exp(e):
    return jnp.exp(jnp.where(e > 0.0, e, 0.2 * e))


def _tc1_body(x_ref, a0_ref, a1_ref, a2_ref, w0_ref, w1_ref, w2_ref, out_ref):
    dn = (((1,), (1,)), ((), ()))
    pt = lax.dot_general(a0_ref[...], w0_ref[...], dn,
                         preferred_element_type=jnp.float32,
                         precision=lax.Precision.HIGHEST)
    pt = pt + lax.dot_general(a1_ref[...], w1_ref[...], dn,
                              preferred_element_type=jnp.float32,
                              precision=lax.Precision.HIGHEST)
    pt = pt + lax.dot_general(a2_ref[...], w2_ref[...], dn,
                              preferred_element_type=jnp.float32,
                              precision=lax.Precision.HIGHEST)
    out_ref[...] = lax.dot_general(pt, x_ref[...], dn,
                                   preferred_element_type=jnp.float32,
                                   precision=lax.Precision.HIGHEST)


def _tc1(xpad, a0, a1, a2, w0, w1, w2):
    return pl.pallas_call(
        _tc1_body,
        out_shape=jax.ShapeDtypeStruct((8, NP), jnp.float32),
    )(xpad, a0, a1, a2, w0, w1, w2)


def _sc1_body(sdt_hbm, src_hbm, dst_hbm,
              ee_hbm, denp_hbm, self_hbm,
              sd_v, sv0, dv0, sv1, dv1,
              e00, e01, e02, e10, e11, e12,
              sf0, sf1, sf2, zero_v,
              dn0_sh, dn1_sh, dn2_sh,
              sem_in0, sem_in1, sem_out0, sem_out1):
    c = lax.axis_index("c")
    s = lax.axis_index("s")
    wid = c * 16 + s
    sf_bufs = (sf0, sf1, sf2)
    dn_refs = (dn0_sh, dn1_sh, dn2_sh)
    sets = (
        (sv0, dv0, (e00, e01, e02), sem_in0, sem_out0),
        (sv1, dv1, (e10, e11, e12), sem_in1, sem_out1),
    )
    nbase = s * SUBSL
    ebase = wid * EPT

    def in_copies(ch):
        sv, dv, _, sem_in, _ = sets[ch % 2]
        return (
            pltpu.make_async_copy(
                src_hbm.at[pl.ds(ebase + ch * CH, CH)], sv, sem_in),
            pltpu.make_async_copy(
                dst_hbm.at[pl.ds(ebase + ch * CH, CH)], dv, sem_in),
        )

    def out_copies(ch):
        _, dv, ee, _, sem_out = sets[ch % 2]
        return [pltpu.make_async_copy(
            ee[i], ee_hbm.at[pl.ds(i * EP + ebase + ch * CH, CH)], sem_out)
            for i in range(3)]

    pending_in = {0: in_copies(0)}
    for d in pending_in[0]:
        d.start()
    pltpu.sync_copy(sdt_hbm, sd_v)

    def self_body(v, _):
        idx = nbase + v * 16 + _iota16()
        for i in range(3):
            si = plsc.load_gather(sd_v, [idx + (2 * i) * NP])
            di = plsc.load_gather(sd_v, [idx + (2 * i + 1) * NP])
            sf_bufs[i][pl.ds(v * 16, 16)] = _leaky_exp(si + di)
        zero_v[pl.ds(v * 16, 16)] = jnp.zeros((16,), jnp.float32)
        return 0

    lax.fori_loop(0, SUBSL // 16, self_body, 0)

    @pl.when(c == 0)
    def _():
        for i in range(3):
            pltpu.sync_copy(sf_bufs[i], dn_refs[i].at[pl.ds(nbase, SUBSL)])
            pltpu.sync_copy(sf_bufs[i],
                            self_hbm.at[pl.ds(i * NP + nbase, SUBSL)])

    @pl.when(c == 1)
    def _():
        for i in range(3):
            pltpu.sync_copy(zero_v, dn_refs[i].at[pl.ds(nbase, SUBSL)])

    plsc.subcore_barrier()

    pending_out = {}
    for ch in range(NCHUNK):
        sv, dv, ee, _, _ = sets[ch % 2]
        for d in pending_in.pop(ch):
            d.wait()
        if ch + 1 < NCHUNK:
            pending_in[ch + 1] = in_copies(ch + 1)
            for d in pending_in[ch + 1]:
                d.start()

        def vec_body(v, _):
            s16 = sv[pl.ds(v * 16, 16)]
            d16 = dv[pl.ds(v * 16, 16)]
            for i in range(3):
                si = plsc.load_gather(sd_v, [s16 + (2 * i) * NP])
                di = plsc.load_gather(sd_v, [d16 + (2 * i + 1) * NP])
                ee[i][pl.ds(v * 16, 16)] = _leaky_exp(si + di)
            return 0

        lax.fori_loop(0, CH // 16, vec_body, 0)

        outs = out_copies(ch)
        for d in outs:
            d.start()
        for i in range(3):
            pltpu.sync_copy(ee[i], dn_refs[i].at[dv], add=True)
        pending_out[ch] = outs
        if ch - 1 >= 0:
            for d in pending_out.pop(ch - 1):
                d.wait()
    for ch in sorted(pending_out):
        for d in pending_out[ch]:
            d.wait()

    plsc.subcore_barrier()
    for i in range(3):
        pltpu.sync_copy(
            dn_refs[i].at[pl.ds(nbase, SUBSL)],
            denp_hbm.at[pl.ds(c * 3 * NP + i * NP + nbase, SUBSL)])


def _sc1(sdt_flat, srcp, dstp):
    f = pl.kernel(
        _sc1_body,
        out_type=(
            jax.ShapeDtypeStruct((3 * EP,), jnp.float32),
            jax.ShapeDtypeStruct((2 * 3 * NP,), jnp.float32),
            jax.ShapeDtypeStruct((3 * NP,), jnp.float32),
        ),
        mesh=_mesh(),
        compiler_params=pltpu.CompilerParams(needs_layout_passes=False),
        scratch_types=[
            pltpu.VMEM((8 * NP,), jnp.float32),
        ] + [pltpu.VMEM((CH,), jnp.int32)] * 4
        + [pltpu.VMEM((CH,), jnp.float32)] * 6
        + [pltpu.VMEM((SUBSL,), jnp.float32)] * 4
        + [pltpu.VMEM_SHARED((NP,), jnp.float32)] * 3
        + [pltpu.SemaphoreType.DMA] * 4,
    )
    return f(sdt_flat, srcp, dstp)


def _sc2_body(src_hbm, dst_hbm, ee_hbm, denp_hbm, self_hbm,
              wsp_hbm,
              dn_v, dm_v,
              sv0, dv0, sv1, dv1,
              e00, e01, e02, e10, e11, e12,
              a00, a01, a02, a10, a11, a12,
              sf0, sf1, sf2, zero_v,
              ws0_sh, ws1_sh, ws2_sh,
              sem_in0, sem_in1, sem_out0, sem_out1, sem_dn):
    c = lax.axis_index("c")
    s = lax.axis_index("s")
    wid = c * 16 + s
    sf_bufs = (sf0, sf1, sf2)
    ws_refs = (ws0_sh, ws1_sh, ws2_sh)
    sets = (
        (sv0, dv0, (e00, e01, e02), (a00, a01, a02), sem_in0, sem_out0),
        (sv1, dv1, (e10, e11, e12), (a10, a11, a12), sem_in1, sem_out1),
    )
    nbase = s * SUBSL
    ebase = wid * EPT

    def in_copies(ch):
        sv, dv, ee, _, sem_in, _ = sets[ch % 2]
        cps = [
            pltpu.make_async_copy(
                src_hbm.at[pl.ds(ebase + ch * CH, CH)], sv, sem_in),
            pltpu.make_async_copy(
                dst_hbm.at[pl.ds(ebase + ch * CH, CH)], dv, sem_in),
        ]
        for i in range(3):
            cps.append(pltpu.make_async_copy(
                ee_hbm.at[pl.ds(i * EP + ebase + ch * CH, CH)], ee[i],
                sem_in))
        return cps

    pending_in = {0: in_copies(0)}
    for d in pending_in[0]:
        d.start()
    dn_cp = pltpu.make_async_copy(denp_hbm.at[pl.ds(0, 3 * NP)], dn_v, sem_dn)
    dm_cp = pltpu.make_async_copy(denp_hbm.at[pl.ds(3 * NP, 3 * NP)], dm_v,
                                  sem_dn)
    dn_cp.start()
    dm_cp.start()
    for i in range(3):
        pltpu.sync_copy(self_hbm.at[pl.ds(i * NP + nbase, SUBSL)], sf_bufs[i])
    dn_cp.wait()
    dm_cp.wait()

    def self_body(v, _):
        o = v * 16
        for i in range(3):
            d16 = (dn_v[pl.ds(i * NP + nbase + o, 16)]
                   + dm_v[pl.ds(i * NP + nbase + o, 16)] + 1e-16)
            sf_bufs[i][pl.ds(o, 16)] = sf_bufs[i][pl.ds(o, 16)] / d16
        zero_v[pl.ds(o, 16)] = jnp.zeros((16,), jnp.float32)
        return 0

    lax.fori_loop(0, SUBSL // 16, self_body, 0)

    @pl.when(c == 0)
    def _():
        for i in range(3):
            pltpu.sync_copy(sf_bufs[i], ws_refs[i].at[pl.ds(nbase, SUBSL)])

    @pl.when(c == 1)
    def _():
        for i in range(3):
            pltpu.sync_copy(zero_v, ws_refs[i].at[pl.ds(nbase, SUBSL)])

    plsc.subcore_barrier()

    for ch in range(NCHUNK):
        sv, dv, ee, al, _, _ = sets[ch % 2]
        for d in pending_in.pop(ch):
            d.wait()
        if ch + 1 < NCHUNK:
            pending_in[ch + 1] = in_copies(ch + 1)
            for d in pending_in[ch + 1]:
                d.start()

        def vec_body(v, _):
            d16 = dv[pl.ds(v * 16, 16)]
            for i in range(3):
                g0 = plsc.load_gather(dn_v, [d16 + i * NP])
                g1 = plsc.load_gather(dm_v, [d16 + i * NP])
                al[i][pl.ds(v * 16, 16)] = (
                    ee[i][pl.ds(v * 16, 16)] / (g0 + g1 + 1e-16)
                )
            return 0

        lax.fori_loop(0, CH // 16, vec_body, 0)

        for i in range(3):
            pltpu.sync_copy(al[i], ws_refs[i].at[sv], add=True)

    plsc.subcore_barrier()
    for i in range(3):
        pltpu.sync_copy(
            ws_refs[i].at[pl.ds(nbase, SUBSL)],
            wsp_hbm.at[pl.ds(c * 3 * NP + i * NP + nbase, SUBSL)])


def _sc2(srcp, dstp, ee, denp, selfee):
    f = pl.kernel(
        _sc2_body,
        out_type=jax.ShapeDtypeStruct((2 * 3 * NP,), jnp.float32),
        mesh=_mesh(),
        compiler_params=pltpu.CompilerParams(needs_layout_passes=False),
        scratch_types=[
            pltpu.VMEM((3 * NP,), jnp.float32),
            pltpu.VMEM((3 * NP,), jnp.float32),
        ] + [pltpu.VMEM((CH,), jnp.int32)] * 4
        + [pltpu.VMEM((CH,), jnp.float32)] * 6
        + [pltpu.VMEM((CH,), jnp.float32)] * 6
        + [pltpu.VMEM((SUBSL,), jnp.float32)] * 4
        + [pltpu.VMEM_SHARED((NP,), jnp.float32)] * 3
        + [pltpu.SemaphoreType.DMA] * 5,
    )
    return f(srcp, dstp, ee, denp, selfee)


def _tc2_body(wsp_ref, x_ref, w0_ref, w1_ref, w2_ref,
              b0_ref, b1_ref, b2_ref, wp_ref, bp_ref, out_ref):
    w = wsp_ref[0] + wsp_ref[1]
    g = lax.dot_general(w, x_ref[...], (((1,), (0,)), ((), ())),
                        preferred_element_type=jnp.float32,
                        precision=lax.Precision.HIGHEST)
    inv_n = 1.0 / N
    m0 = jnp.dot(g[0:1], w0_ref[...], preferred_element_type=jnp.float32,
                 precision=lax.Precision.HIGHEST) * inv_n + b0_ref[...]
    m1 = jnp.dot(g[1:2], w1_ref[...], preferred_element_type=jnp.float32,
                 precision=lax.Precision.HIGHEST) * inv_n + b1_ref[...]
    m2 = jnp.dot(g[2:3], w2_ref[...], preferred_element_type=jnp.float32,
                 precision=lax.Precision.HIGHEST) * inv_n + b2_ref[...]
    cm = jnp.concatenate([m0, m1, m2], axis=1)
    out_ref[...] = jnp.dot(cm, wp_ref[...], preferred_element_type=jnp.float32,
                           precision=lax.Precision.HIGHEST) + bp_ref[...]


def _tc2(wsp, xpad, w0, w1, w2, b0, b1, b2, wp, bp):
    return pl.pallas_call(
        _tc2_body,
        out_shape=jax.ShapeDtypeStruct((1, D), jnp.float32),
    )(wsp, xpad, w0, w1, w2, b0, b1, b2, wp, bp)


@jax.jit
def kernel(x, edge_index, W0, a_src0, a_dst0, b0, W1, a_src1, a_dst1, b1,
           W2, a_src2, a_dst2, b2, Wp, bp):
    xpad = jnp.pad(x, ((0, NP - N), (0, 0)))
    pad = jnp.full((EP - E,), N, dtype=jnp.int32)
    srcp = jnp.concatenate([edge_index[0].astype(jnp.int32), pad])
    dstp = jnp.concatenate([edge_index[1].astype(jnp.int32), pad])

    def amat(a_s, a_d, i):
        z = jnp.zeros((8, D), jnp.float32)
        return z.at[2 * i].set(a_s).at[2 * i + 1].set(a_d)

    a0 = amat(a_src0, a_dst0, 0)
    a1 = amat(a_src1, a_dst1, 1)
    a2 = amat(a_src2, a_dst2, 2)

    sdt = _tc1(xpad, a0, a1, a2, W0, W1, W2)
    sdt_flat = sdt.reshape(-1)

    ee, denp, selfee = _sc1(sdt_flat, srcp, dstp)
    wsp = _sc2(srcp, dstp, ee, denp, selfee)

    return _tc2(wsp.reshape(2, 3, NP), xpad, W0, W1, W2,
                b0.reshape(1, D), b1.reshape(1, D), b2.reshape(1, D),
                Wp, bp.reshape(1, D))


# speedup vs baseline: 1.3134x; 1.3134x over previous
import jax
import jax.numpy as jnp
from jax import lax
from jax.experimental import pallas as pl
from jax.experimental.pallas import tpu as pltpu
from jax.experimental.pallas import tpu_sc as plsc

N = 10000
NP = 10240
D = 128
E = 320000
NTILES = 32
EPT = 10000
NCHUNK = 5
CH = 2000
SUBSL = NP // 16


def _mesh():
    return plsc.VectorSubcoreMesh(
        core_axis_name="c", subcore_axis_name="s", num_cores=2, num_subcores=16
    )


def _iota16():
    return lax.broadcasted_iota(jnp.int32, (16,), 0)


def _leaky_exp(e):
    return jnp.exp(jnp.where(e > 0.0, e, 0.2 * e))


def _tc1_body(x_ref, a0_ref, a1_ref, a2_ref, w0_ref, w1_ref, w2_ref, out_ref):
    dn = (((1,), (1,)), ((), ()))
    pt = lax.dot_general(a0_ref[...], w0_ref[...], dn,
                         preferred_element_type=jnp.float32,
                         precision=lax.Precision.HIGHEST)
    pt = pt + lax.dot_general(a1_ref[...], w1_ref[...], dn,
                              preferred_element_type=jnp.float32,
                              precision=lax.Precision.HIGHEST)
    pt = pt + lax.dot_general(a2_ref[...], w2_ref[...], dn,
                              preferred_element_type=jnp.float32,
                              precision=lax.Precision.HIGHEST)
    out_ref[...] = lax.dot_general(pt, x_ref[...], dn,
                                   preferred_element_type=jnp.float32,
                                   precision=lax.Precision.HIGHEST)


def _tc1(xpad, a0, a1, a2, w0, w1, w2):
    return pl.pallas_call(
        _tc1_body,
        out_shape=jax.ShapeDtypeStruct((8, NP), jnp.float32),
    )(xpad, a0, a1, a2, w0, w1, w2)


def _sc1_body(sdt_hbm, edge_hbm,
              ee_hbm, denp_hbm, self_hbm,
              sd_v, sv0, dv0, sv1, dv1,
              e00, e01, e02, e10, e11, e12,
              sf0, sf1, sf2, zero_v,
              dn0_sh, dn1_sh, dn2_sh,
              sem_in0, sem_in1, sem_out0, sem_out1):
    c = lax.axis_index("c")
    s = lax.axis_index("s")
    wid = c * 16 + s
    sf_bufs = (sf0, sf1, sf2)
    dn_refs = (dn0_sh, dn1_sh, dn2_sh)
    sets = (
        (sv0, dv0, (e00, e01, e02), sem_in0, sem_out0),
        (sv1, dv1, (e10, e11, e12), sem_in1, sem_out1),
    )
    nbase = s * SUBSL
    ebase = wid * EPT

    def in_copies(ch):
        sv, dv, _, sem_in, _ = sets[ch % 2]
        return (
            pltpu.make_async_copy(
                edge_hbm.at[pl.ds(ebase + ch * CH, CH)], sv, sem_in),
            pltpu.make_async_copy(
                edge_hbm.at[pl.ds(E + ebase + ch * CH, CH)], dv, sem_in),
        )

    def out_copies(ch):
        _, dv, ee, _, sem_out = sets[ch % 2]
        return [pltpu.make_async_copy(
            ee[i], ee_hbm.at[pl.ds(i * E + ebase + ch * CH, CH)], sem_out)
            for i in range(3)]

    pending_in = {0: in_copies(0)}
    for d in pending_in[0]:
        d.start()
    pltpu.sync_copy(sdt_hbm, sd_v)

    def self_body(v, _):
        idx = nbase + v * 16 + _iota16()
        for i in range(3):
            si = plsc.load_gather(sd_v, [idx + (2 * i) * NP])
            di = plsc.load_gather(sd_v, [idx + (2 * i + 1) * NP])
            sf_bufs[i][pl.ds(v * 16, 16)] = _leaky_exp(si + di)
        zero_v[pl.ds(v * 16, 16)] = jnp.zeros((16,), jnp.float32)
        return 0

    lax.fori_loop(0, SUBSL // 16, self_body, 0)

    @pl.when(c == 0)
    def _():
        for i in range(3):
            pltpu.sync_copy(sf_bufs[i], dn_refs[i].at[pl.ds(nbase, SUBSL)])
            pltpu.sync_copy(sf_bufs[i],
                            self_hbm.at[pl.ds(i * NP + nbase, SUBSL)])

    @pl.when(c == 1)
    def _():
        for i in range(3):
            pltpu.sync_copy(zero_v, dn_refs[i].at[pl.ds(nbase, SUBSL)])

    plsc.subcore_barrier()

    pending_out = {}
    for ch in range(NCHUNK):
        sv, dv, ee, _, _ = sets[ch % 2]
        for d in pending_in.pop(ch):
            d.wait()
        if ch + 1 < NCHUNK:
            pending_in[ch + 1] = in_copies(ch + 1)
            for d in pending_in[ch + 1]:
                d.start()

        def vec_body(v, _):
            s16 = sv[pl.ds(v * 16, 16)]
            d16 = dv[pl.ds(v * 16, 16)]
            for i in range(3):
                si = plsc.load_gather(sd_v, [s16 + (2 * i) * NP])
                di = plsc.load_gather(sd_v, [d16 + (2 * i + 1) * NP])
                ee[i][pl.ds(v * 16, 16)] = _leaky_exp(si + di)
            return 0

        lax.fori_loop(0, CH // 16, vec_body, 0)

        outs = out_copies(ch)
        for d in outs:
            d.start()
        for i in range(3):
            pltpu.sync_copy(ee[i], dn_refs[i].at[dv], add=True)
        pending_out[ch] = outs
        if ch - 1 >= 0:
            for d in pending_out.pop(ch - 1):
                d.wait()
    for ch in sorted(pending_out):
        for d in pending_out[ch]:
            d.wait()

    plsc.subcore_barrier()
    for i in range(3):
        pltpu.sync_copy(
            dn_refs[i].at[pl.ds(nbase, SUBSL)],
            denp_hbm.at[pl.ds(c * 3 * NP + i * NP + nbase, SUBSL)])


def _sc1(sdt_flat, eflat):
    f = pl.kernel(
        _sc1_body,
        out_type=(
            jax.ShapeDtypeStruct((3 * E,), jnp.float32),
            jax.ShapeDtypeStruct((2 * 3 * NP,), jnp.float32),
            jax.ShapeDtypeStruct((3 * NP,), jnp.float32),
        ),
        mesh=_mesh(),
        compiler_params=pltpu.CompilerParams(needs_layout_passes=False),
        scratch_types=[
            pltpu.VMEM((8 * NP,), jnp.float32),
        ] + [pltpu.VMEM((CH,), jnp.int32)] * 4
        + [pltpu.VMEM((CH,), jnp.float32)] * 6
        + [pltpu.VMEM((SUBSL,), jnp.float32)] * 4
        + [pltpu.VMEM_SHARED((NP,), jnp.float32)] * 3
        + [pltpu.SemaphoreType.DMA] * 4,
    )
    return f(sdt_flat, eflat)


def _sc2_body(edge_hbm, ee_hbm, denp_hbm, self_hbm,
              wsp_hbm,
              dn_v, dm_v,
              sv0, dv0, sv1, dv1,
              e00, e01, e02, e10, e11, e12,
              a00, a01, a02, a10, a11, a12,
              sf0, sf1, sf2, zero_v,
              ws0_sh, ws1_sh, ws2_sh,
              sem_in0, sem_in1, sem_out0, sem_out1, sem_dn):
    c = lax.axis_index("c")
    s = lax.axis_index("s")
    wid = c * 16 + s
    sf_bufs = (sf0, sf1, sf2)
    ws_refs = (ws0_sh, ws1_sh, ws2_sh)
    sets = (
        (sv0, dv0, (e00, e01, e02), (a00, a01, a02), sem_in0, sem_out0),
        (sv1, dv1, (e10, e11, e12), (a10, a11, a12), sem_in1, sem_out1),
    )
    nbase = s * SUBSL
    ebase = wid * EPT

    def in_copies(ch):
        sv, dv, ee, _, sem_in, _ = sets[ch % 2]
        cps = [
            pltpu.make_async_copy(
                edge_hbm.at[pl.ds(ebase + ch * CH, CH)], sv, sem_in),
            pltpu.make_async_copy(
                edge_hbm.at[pl.ds(E + ebase + ch * CH, CH)], dv, sem_in),
        ]
        for i in range(3):
            cps.append(pltpu.make_async_copy(
                ee_hbm.at[pl.ds(i * E + ebase + ch * CH, CH)], ee[i],
                sem_in))
        return cps

    pending_in = {0: in_copies(0)}
    for d in pending_in[0]:
        d.start()
    dn_cp = pltpu.make_async_copy(denp_hbm.at[pl.ds(0, 3 * NP)], dn_v, sem_dn)
    dm_cp = pltpu.make_async_copy(denp_hbm.at[pl.ds(3 * NP, 3 * NP)], dm_v,
                                  sem_dn)
    dn_cp.start()
    dm_cp.start()
    for i in range(3):
        pltpu.sync_copy(self_hbm.at[pl.ds(i * NP + nbase, SUBSL)], sf_bufs[i])
    dn_cp.wait()
    dm_cp.wait()

    def self_body(v, _):
        o = v * 16
        for i in range(3):
            d16 = (dn_v[pl.ds(i * NP + nbase + o, 16)]
                   + dm_v[pl.ds(i * NP + nbase + o, 16)] + 1e-16)
            sf_bufs[i][pl.ds(o, 16)] = sf_bufs[i][pl.ds(o, 16)] / d16
        zero_v[pl.ds(o, 16)] = jnp.zeros((16,), jnp.float32)
        return 0

    lax.fori_loop(0, SUBSL // 16, self_body, 0)

    @pl.when(c == 0)
    def _():
        for i in range(3):
            pltpu.sync_copy(sf_bufs[i], ws_refs[i].at[pl.ds(nbase, SUBSL)])

    @pl.when(c == 1)
    def _():
        for i in range(3):
            pltpu.sync_copy(zero_v, ws_refs[i].at[pl.ds(nbase, SUBSL)])

    plsc.subcore_barrier()

    for ch in range(NCHUNK):
        sv, dv, ee, al, _, _ = sets[ch % 2]
        for d in pending_in.pop(ch):
            d.wait()
        if ch + 1 < NCHUNK:
            pending_in[ch + 1] = in_copies(ch + 1)
            for d in pending_in[ch + 1]:
                d.start()

        def vec_body(v, _):
            d16 = dv[pl.ds(v * 16, 16)]
            for i in range(3):
                g0 = plsc.load_gather(dn_v, [d16 + i * NP])
                g1 = plsc.load_gather(dm_v, [d16 + i * NP])
                al[i][pl.ds(v * 16, 16)] = (
                    ee[i][pl.ds(v * 16, 16)] / (g0 + g1 + 1e-16)
                )
            return 0

        lax.fori_loop(0, CH // 16, vec_body, 0)

        for i in range(3):
            pltpu.sync_copy(al[i], ws_refs[i].at[sv], add=True)

    plsc.subcore_barrier()
    for i in range(3):
        pltpu.sync_copy(
            ws_refs[i].at[pl.ds(nbase, SUBSL)],
            wsp_hbm.at[pl.ds(c * 3 * NP + i * NP + nbase, SUBSL)])


def _sc2(eflat, ee, denp, selfee):
    f = pl.kernel(
        _sc2_body,
        out_type=jax.ShapeDtypeStruct((2 * 3 * NP,), jnp.float32),
        mesh=_mesh(),
        compiler_params=pltpu.CompilerParams(needs_layout_passes=False),
        scratch_types=[
            pltpu.VMEM((3 * NP,), jnp.float32),
            pltpu.VMEM((3 * NP,), jnp.float32),
        ] + [pltpu.VMEM((CH,), jnp.int32)] * 4
        + [pltpu.VMEM((CH,), jnp.float32)] * 6
        + [pltpu.VMEM((CH,), jnp.float32)] * 6
        + [pltpu.VMEM((SUBSL,), jnp.float32)] * 4
        + [pltpu.VMEM_SHARED((NP,), jnp.float32)] * 3
        + [pltpu.SemaphoreType.DMA] * 5,
    )
    return f(eflat, ee, denp, selfee)


def _tc2_body(wsp_ref, x_ref, w0_ref, w1_ref, w2_ref,
              b0_ref, b1_ref, b2_ref, wp_ref, bp_ref, out_ref):
    w = wsp_ref[0] + wsp_ref[1]
    g = lax.dot_general(w, x_ref[...], (((1,), (0,)), ((), ())),
                        preferred_element_type=jnp.float32,
                        precision=lax.Precision.HIGHEST)
    inv_n = 1.0 / N
    m0 = jnp.dot(g[0:1], w0_ref[...], preferred_element_type=jnp.float32,
                 precision=lax.Precision.HIGHEST) * inv_n + b0_ref[...]
    m1 = jnp.dot(g[1:2], w1_ref[...], preferred_element_type=jnp.float32,
                 precision=lax.Precision.HIGHEST) * inv_n + b1_ref[...]
    m2 = jnp.dot(g[2:3], w2_ref[...], preferred_element_type=jnp.float32,
                 precision=lax.Precision.HIGHEST) * inv_n + b2_ref[...]
    cm = jnp.concatenate([m0, m1, m2], axis=1)
    out_ref[...] = jnp.dot(cm, wp_ref[...], preferred_element_type=jnp.float32,
                           precision=lax.Precision.HIGHEST) + bp_ref[...]


def _tc2(wsp, xpad, w0, w1, w2, b0, b1, b2, wp, bp):
    return pl.pallas_call(
        _tc2_body,
        out_shape=jax.ShapeDtypeStruct((1, D), jnp.float32),
    )(wsp, xpad, w0, w1, w2, b0, b1, b2, wp, bp)


@jax.jit
def kernel(x, edge_index, W0, a_src0, a_dst0, b0, W1, a_src1, a_dst1, b1,
           W2, a_src2, a_dst2, b2, Wp, bp):
    xpad = jnp.pad(x, ((0, NP - N), (0, 0)))
    eflat = edge_index.astype(jnp.int32).reshape(-1)

    def amat(a_s, a_d, i):
        z = jnp.zeros((8, D), jnp.float32)
        return z.at[2 * i].set(a_s).at[2 * i + 1].set(a_d)

    a0 = amat(a_src0, a_dst0, 0)
    a1 = amat(a_src1, a_dst1, 1)
    a2 = amat(a_src2, a_dst2, 2)

    sdt = _tc1(xpad, a0, a1, a2, W0, W1, W2)
    sdt_flat = sdt.reshape(-1)

    ee, denp, selfee = _sc1(sdt_flat, eflat)
    wsp = _sc2(eflat, ee, denp, selfee)

    return _tc2(wsp.reshape(2, 3, NP), xpad, W0, W1, W2,
                b0.reshape(1, D), b1.reshape(1, D), b2.reshape(1, D),
                Wp, bp.reshape(1, D))
